# baseline (speedup 1.0000x reference)
import dataclasses
import functools

import jax
import jax.numpy as jnp
from jax import lax
from jax.experimental import pallas as pl
from jax.experimental.pallas import tpu as pltpu
from jax.experimental.pallas import tpu_sc as plsc

VOCAB = 13
L = 200
B = 16384
E = 32
H = 16
O = 4

NUM_TILES = 32
ROWS_PER_TILE = B // NUM_TILES
CHUNK_ROWS = 64
NUM_CHUNKS = ROWS_PER_TILE // CHUNK_ROWS
CHUNK_ELEMS = CHUNK_ROWS * L
CHUNK_BINS = CHUNK_ROWS * 16


def _sc_hist_kernel(x_hbm, offs_hbm, cnt_hbm,
                    xv0, xv1, offs_v, cnt0, cnt1,
                    s_in0, s_in1, s_out0, s_out1, s_offs):
    wid = lax.axis_index("s") * 2 + lax.axis_index("c")
    base = wid * (ROWS_PER_TILE * L)
    cnt_base = wid * (ROWS_PER_TILE * 16)

    ones = jnp.full((16,), 1.0, dtype=jnp.float32)

    pltpu.async_copy(offs_hbm, offs_v, s_offs).wait()

    xbufs = [xv0, xv1]
    cbufs = [cnt0, cnt1]
    in_sems = [s_in0, s_in1]
    out_sems = [s_out0, s_out1]

    in_copies = [None, None]
    out_copies = [None, None]

    in_copies[0] = pltpu.async_copy(
        x_hbm.at[pl.ds(base, CHUNK_ELEMS)], xbufs[0], in_sems[0])

    for ch in range(NUM_CHUNKS):
        p = ch % 2
        if ch + 1 < NUM_CHUNKS:
            in_copies[1 - p] = pltpu.async_copy(
                x_hbm.at[pl.ds(base + (ch + 1) * CHUNK_ELEMS, CHUNK_ELEMS)],
                xbufs[1 - p], in_sems[1 - p])
        if out_copies[p] is not None:
            out_copies[p].wait()
        cntv = cbufs[p]

        @pl.loop(0, CHUNK_BINS, step=16)
        def _zero(j, cntv=cntv):
            cntv[pl.ds(j, 16)] = jnp.zeros((16,), jnp.float32)

        in_copies[p].wait()
        xv = xbufs[p]

        @pl.loop(0, CHUNK_ELEMS, step=16)
        def _scatter(i, xv=xv, cntv=cntv):
            idx = xv[pl.ds(i, 16)] + offs_v[pl.ds(i, 16)]
            plsc.addupdate_scatter(cntv, [idx], ones)

        out_copies[p] = pltpu.async_copy(
            cntv, cnt_hbm.at[pl.ds(cnt_base + ch * CHUNK_BINS, CHUNK_BINS)],
            out_sems[p])

    for p in range(2):
        if out_copies[p] is not None:
            out_copies[p].wait()


def _sc_histogram(x_flat, offs):
    mesh = plsc.VectorSubcoreMesh(core_axis_name="c", subcore_axis_name="s")
    cp = pltpu.CompilerParams()
    if "needs_layout_passes" in pltpu.CompilerParams.__dataclass_fields__:
        cp = dataclasses.replace(cp, needs_layout_passes=False)
    f = pl.kernel(
        _sc_hist_kernel,
        out_type=jax.ShapeDtypeStruct((B * 16,), jnp.float32),
        mesh=mesh,
        scratch_types=[
            pltpu.VMEM((CHUNK_ELEMS,), jnp.int32),
            pltpu.VMEM((CHUNK_ELEMS,), jnp.int32),
            pltpu.VMEM((CHUNK_ELEMS,), jnp.int32),
            pltpu.VMEM((CHUNK_BINS,), jnp.float32),
            pltpu.VMEM((CHUNK_BINS,), jnp.float32),
            pltpu.SemaphoreType.DMA,
            pltpu.SemaphoreType.DMA,
            pltpu.SemaphoreType.DMA,
            pltpu.SemaphoreType.DMA,
            pltpu.SemaphoreType.DMA,
        ],
        compiler_params=cp,
    )
    return f(x_flat, offs)


MLP_BLK = 2048


def _mlp_kernel(cnt_ref, table_ref, w1_ref, b1_ref, w2_ref, b2_ref, out_ref):
    m = jnp.dot(table_ref[...], w1_ref[...].T,
                preferred_element_type=jnp.float32) * (1.0 / L)
    mp = jnp.concatenate([m, jnp.zeros((16 - VOCAB, H), jnp.float32)], axis=0)
    cnt = cnt_ref[...]
    h = jnp.maximum(
        jnp.dot(cnt, mp, preferred_element_type=jnp.float32) + b1_ref[0][None, :],
        0.0)
    out_ref[...] = (jnp.dot(h, w2_ref[...].T, preferred_element_type=jnp.float32)
                    + b2_ref[0][None, :])


def _tc_mlp(counts, table, W1, b1, W2, b2):
    return pl.pallas_call(
        _mlp_kernel,
        grid=(B // MLP_BLK,),
        in_specs=[
            pl.BlockSpec((MLP_BLK, 16), lambda i: (i, 0)),
            pl.BlockSpec((VOCAB, E), lambda i: (0, 0)),
            pl.BlockSpec((H, E), lambda i: (0, 0)),
            pl.BlockSpec((1, H), lambda i: (0, 0)),
            pl.BlockSpec((O, H), lambda i: (0, 0)),
            pl.BlockSpec((1, O), lambda i: (0, 0)),
        ],
        out_specs=pl.BlockSpec((MLP_BLK, O), lambda i: (i, 0)),
        out_shape=jax.ShapeDtypeStruct((B, O), jnp.float32),
        compiler_params=pltpu.CompilerParams(
            dimension_semantics=("arbitrary",),
        ),
    )(counts, table, W1, b1.reshape(1, H), W2, b2.reshape(1, O))


def kernel(x, table, W1, b1, W2, b2):
    x_flat = x.astype(jnp.int32).reshape(B * L)
    offs = (jnp.arange(CHUNK_ELEMS, dtype=jnp.int32) // L) * 16
    counts_flat = _sc_histogram(x_flat, offs)
    counts = counts_flat.reshape(B, 16)
    return _tc_mlp(counts, table, W1, b1, W2, b2)


# speedup vs baseline: 1.3348x; 1.3348x over previous
import dataclasses
import functools

import jax
import jax.numpy as jnp
from jax import lax
from jax.experimental import pallas as pl
from jax.experimental.pallas import tpu as pltpu
from jax.experimental.pallas import tpu_sc as plsc

VOCAB = 13
L = 200
B = 16384
E = 32
H = 16
O = 4

NUM_TILES = 32
ROWS_PER_TILE = B // NUM_TILES
CHUNK_ROWS = 64
NUM_CHUNKS = ROWS_PER_TILE // CHUNK_ROWS
CHUNK_ELEMS = CHUNK_ROWS * L
CHUNK_BINS = CHUNK_ROWS * 16


def _sc_hist_kernel(x_hbm, offs_hbm, cnt_hbm,
                    xv0, xv1, offs_v, cnt0, cnt1,
                    s_in0, s_in1, s_out0, s_out1, s_offs):
    wid = lax.axis_index("s") * 2 + lax.axis_index("c")
    base = wid * (ROWS_PER_TILE * L)
    cnt_base = wid * (ROWS_PER_TILE * 16)

    ones = jnp.full((16,), 1.0, dtype=jnp.float32)

    pltpu.async_copy(offs_hbm, offs_v, s_offs).wait()

    xbufs = [xv0, xv1]
    cbufs = [cnt0, cnt1]
    in_sems = [s_in0, s_in1]
    out_sems = [s_out0, s_out1]

    in_copies = [None, None]
    out_copies = [None, None]

    in_copies[0] = pltpu.async_copy(
        x_hbm.at[pl.ds(base, CHUNK_ELEMS)], xbufs[0], in_sems[0])

    for ch in range(NUM_CHUNKS):
        p = ch % 2
        if ch + 1 < NUM_CHUNKS:
            in_copies[1 - p] = pltpu.async_copy(
                x_hbm.at[pl.ds(base + (ch + 1) * CHUNK_ELEMS, CHUNK_ELEMS)],
                xbufs[1 - p], in_sems[1 - p])
        if out_copies[p] is not None:
            out_copies[p].wait()
        cntv = cbufs[p]

        @plsc.parallel_loop(0, CHUNK_BINS, step=16, unroll=8)
        def _zero(j, cntv=cntv):
            cntv[pl.ds(j, 16)] = jnp.zeros((16,), jnp.float32)

        in_copies[p].wait()
        xv = xbufs[p]

        @plsc.parallel_loop(0, CHUNK_ELEMS, step=16, unroll=8)
        def _scatter(i, xv=xv, cntv=cntv):
            idx = xv[pl.ds(i, 16)] + offs_v[pl.ds(i, 16)]
            plsc.addupdate_scatter(cntv, [idx], ones)

        out_copies[p] = pltpu.async_copy(
            cntv, cnt_hbm.at[pl.ds(cnt_base + ch * CHUNK_BINS, CHUNK_BINS)],
            out_sems[p])

    for p in range(2):
        if out_copies[p] is not None:
            out_copies[p].wait()


def _sc_histogram(x_flat, offs):
    mesh = plsc.VectorSubcoreMesh(core_axis_name="c", subcore_axis_name="s")
    cp = pltpu.CompilerParams()
    if "needs_layout_passes" in pltpu.CompilerParams.__dataclass_fields__:
        cp = dataclasses.replace(cp, needs_layout_passes=False)
    f = pl.kernel(
        _sc_hist_kernel,
        out_type=jax.ShapeDtypeStruct((B * 16,), jnp.float32),
        mesh=mesh,
        scratch_types=[
            pltpu.VMEM((CHUNK_ELEMS,), jnp.int32),
            pltpu.VMEM((CHUNK_ELEMS,), jnp.int32),
            pltpu.VMEM((CHUNK_ELEMS,), jnp.int32),
            pltpu.VMEM((CHUNK_BINS,), jnp.float32),
            pltpu.VMEM((CHUNK_BINS,), jnp.float32),
            pltpu.SemaphoreType.DMA,
            pltpu.SemaphoreType.DMA,
            pltpu.SemaphoreType.DMA,
            pltpu.SemaphoreType.DMA,
            pltpu.SemaphoreType.DMA,
        ],
        compiler_params=cp,
    )
    return f(x_flat, offs)


MLP_BLK = 2048


def _mlp_kernel(cnt_ref, table_ref, w1_ref, b1_ref, w2_ref, b2_ref, out_ref):
    m = jnp.dot(table_ref[...], w1_ref[...].T,
                preferred_element_type=jnp.float32) * (1.0 / L)
    mp = jnp.concatenate([m, jnp.zeros((16 - VOCAB, H), jnp.float32)], axis=0)
    cnt = cnt_ref[...]
    h = jnp.maximum(
        jnp.dot(cnt, mp, preferred_element_type=jnp.float32) + b1_ref[0][None, :],
        0.0)
    out_ref[...] = (jnp.dot(h, w2_ref[...].T, preferred_element_type=jnp.float32)
                    + b2_ref[0][None, :])


def _tc_mlp(counts, table, W1, b1, W2, b2):
    return pl.pallas_call(
        _mlp_kernel,
        grid=(B // MLP_BLK,),
        in_specs=[
            pl.BlockSpec((MLP_BLK, 16), lambda i: (i, 0)),
            pl.BlockSpec((VOCAB, E), lambda i: (0, 0)),
            pl.BlockSpec((H, E), lambda i: (0, 0)),
            pl.BlockSpec((1, H), lambda i: (0, 0)),
            pl.BlockSpec((O, H), lambda i: (0, 0)),
            pl.BlockSpec((1, O), lambda i: (0, 0)),
        ],
        out_specs=pl.BlockSpec((MLP_BLK, O), lambda i: (i, 0)),
        out_shape=jax.ShapeDtypeStruct((B, O), jnp.float32),
        compiler_params=pltpu.CompilerParams(
            dimension_semantics=("arbitrary",),
        ),
    )(counts, table, W1, b1.reshape(1, H), W2, b2.reshape(1, O))


def kernel(x, table, W1, b1, W2, b2):
    x_flat = x.astype(jnp.int32).reshape(B * L)
    offs = (jnp.arange(CHUNK_ELEMS, dtype=jnp.int32) // L) * 16
    counts_flat = _sc_histogram(x_flat, offs)
    counts = counts_flat.reshape(B, 16)
    return _tc_mlp(counts, table, W1, b1, W2, b2)


# speedup vs baseline: 2.0303x; 1.5211x over previous
import dataclasses

import jax
import jax.numpy as jnp
from jax import lax
from jax.experimental import pallas as pl
from jax.experimental.pallas import tpu as pltpu
from jax.experimental.pallas import tpu_sc as plsc

VOCAB = 13
L = 200
B = 16384
E = 32
H = 16
O = 4

NUM_TILES = 32
ROWS_PER_TILE = B // NUM_TILES
CHUNK_ROWS = 128
NUM_CHUNKS = ROWS_PER_TILE // CHUNK_ROWS
FULL_VECS = L // 16
TAIL_OFF = L - 16
TAIL_MASKED = 16 - (L - 16 * FULL_VECS)


def _sc_hist_kernel(x_hbm, cnt_hbm,
                    xv0, xv1, cnt0, cnt1,
                    s_in0, s_in1, s_out0, s_out1):
    wid = lax.axis_index("s") * 2 + lax.axis_index("c")
    row_base = wid * ROWS_PER_TILE

    ones = jnp.full((16,), 1.0, dtype=jnp.float32)
    zeros = jnp.zeros((16,), dtype=jnp.float32)
    tail_mask = jnp.arange(16, dtype=jnp.int32) >= TAIL_MASKED

    xbufs = [xv0, xv1]
    cbufs = [cnt0, cnt1]
    in_sems = [s_in0, s_in1]
    out_sems = [s_out0, s_out1]

    in_copies = [None, None]
    out_copies = [None, None]

    in_copies[0] = pltpu.async_copy(
        x_hbm.at[pl.ds(row_base, CHUNK_ROWS)], xbufs[0], in_sems[0])

    for ch in range(NUM_CHUNKS):
        p = ch % 2
        if ch + 1 < NUM_CHUNKS:
            in_copies[1 - p] = pltpu.async_copy(
                x_hbm.at[pl.ds(row_base + (ch + 1) * CHUNK_ROWS, CHUNK_ROWS)],
                xbufs[1 - p], in_sems[1 - p])
        if out_copies[p] is not None:
            out_copies[p].wait()
        in_copies[p].wait()
        xv = xbufs[p]
        cntv = cbufs[p]

        @plsc.parallel_loop(0, CHUNK_ROWS, step=1, unroll=2)
        def _row(r, xv=xv, cntv=cntv):
            cntv[r, pl.ds(0, 16)] = zeros
            ridx = jnp.full((16,), r, dtype=jnp.int32)
            for k in range(FULL_VECS):
                vals = xv[r, pl.ds(k * 16, 16)]
                plsc.addupdate_scatter(cntv, [ridx, vals], ones)
            tail = xv[r, pl.ds(TAIL_OFF, 16)]
            plsc.addupdate_scatter(cntv, [ridx, tail], ones, mask=tail_mask)

        out_copies[p] = pltpu.async_copy(
            cntv, cnt_hbm.at[pl.ds(row_base + ch * CHUNK_ROWS, CHUNK_ROWS)],
            out_sems[p])

    for p in range(2):
        if out_copies[p] is not None:
            out_copies[p].wait()


def _sc_histogram(x):
    mesh = plsc.VectorSubcoreMesh(core_axis_name="c", subcore_axis_name="s")
    cp = pltpu.CompilerParams()
    if "needs_layout_passes" in pltpu.CompilerParams.__dataclass_fields__:
        cp = dataclasses.replace(cp, needs_layout_passes=False)
    f = pl.kernel(
        _sc_hist_kernel,
        out_type=jax.ShapeDtypeStruct((B, 16), jnp.float32),
        mesh=mesh,
        scratch_types=[
            pltpu.VMEM((CHUNK_ROWS, L), jnp.int32),
            pltpu.VMEM((CHUNK_ROWS, L), jnp.int32),
            pltpu.VMEM((CHUNK_ROWS, 16), jnp.float32),
            pltpu.VMEM((CHUNK_ROWS, 16), jnp.float32),
            pltpu.SemaphoreType.DMA,
            pltpu.SemaphoreType.DMA,
            pltpu.SemaphoreType.DMA,
            pltpu.SemaphoreType.DMA,
        ],
        compiler_params=cp,
    )
    return f(x)


MLP_BLK = 2048


def _mlp_kernel(cnt_ref, table_ref, w1_ref, b1_ref, w2_ref, b2_ref, out_ref):
    m = jnp.dot(table_ref[...], w1_ref[...].T,
                preferred_element_type=jnp.float32) * (1.0 / L)
    mp = jnp.concatenate([m, jnp.zeros((16 - VOCAB, H), jnp.float32)], axis=0)
    cnt = cnt_ref[...]
    h = jnp.maximum(
        jnp.dot(cnt, mp, preferred_element_type=jnp.float32) + b1_ref[0][None, :],
        0.0)
    out_ref[...] = (jnp.dot(h, w2_ref[...].T, preferred_element_type=jnp.float32)
                    + b2_ref[0][None, :])


def _tc_mlp(counts, table, W1, b1, W2, b2):
    return pl.pallas_call(
        _mlp_kernel,
        grid=(B // MLP_BLK,),
        in_specs=[
            pl.BlockSpec((MLP_BLK, 16), lambda i: (i, 0)),
            pl.BlockSpec((VOCAB, E), lambda i: (0, 0)),
            pl.BlockSpec((H, E), lambda i: (0, 0)),
            pl.BlockSpec((1, H), lambda i: (0, 0)),
            pl.BlockSpec((O, H), lambda i: (0, 0)),
            pl.BlockSpec((1, O), lambda i: (0, 0)),
        ],
        out_specs=pl.BlockSpec((MLP_BLK, O), lambda i: (i, 0)),
        out_shape=jax.ShapeDtypeStruct((B, O), jnp.float32),
        compiler_params=pltpu.CompilerParams(
            dimension_semantics=("arbitrary",),
        ),
    )(counts, table, W1, b1.reshape(1, H), W2, b2.reshape(1, O))


def kernel(x, table, W1, b1, W2, b2):
    counts = _sc_histogram(x.astype(jnp.int32))
    return _tc_mlp(counts, table, W1, b1, W2, b2)


# speedup vs baseline: 2.0361x; 1.0028x over previous
import dataclasses

import jax
import jax.numpy as jnp
from jax import lax
from jax.experimental import pallas as pl
from jax.experimental.pallas import tpu as pltpu
from jax.experimental.pallas import tpu_sc as plsc

VOCAB = 13
L = 200
B = 16384
E = 32
H = 16
O = 4

NUM_TILES = 32
ROWS_PER_TILE = B // NUM_TILES
CHUNK_ROWS = 128
NUM_CHUNKS = ROWS_PER_TILE // CHUNK_ROWS
FULL_VECS = L // 16
TAIL_OFF = L - 16
TAIL_MASKED = 16 - (L - 16 * FULL_VECS)


def _sc_hist_kernel(x_hbm, cnt_hbm,
                    xv0, xv1, cnt0, cnt1,
                    s_in0, s_in1, s_out0, s_out1):
    wid = lax.axis_index("s") * 2 + lax.axis_index("c")
    row_base = wid * ROWS_PER_TILE

    ones = jnp.full((16,), 1.0, dtype=jnp.float32)
    zeros = jnp.zeros((16,), dtype=jnp.float32)
    tail_mask = jnp.arange(16, dtype=jnp.int32) >= TAIL_MASKED

    xbufs = [xv0, xv1]
    cbufs = [cnt0, cnt1]
    in_sems = [s_in0, s_in1]
    out_sems = [s_out0, s_out1]

    in_copies = [None, None]
    out_copies = [None, None]

    in_copies[0] = pltpu.async_copy(
        x_hbm.at[pl.ds(row_base, CHUNK_ROWS)], xbufs[0], in_sems[0])

    for ch in range(NUM_CHUNKS):
        p = ch % 2
        if ch + 1 < NUM_CHUNKS:
            in_copies[1 - p] = pltpu.async_copy(
                x_hbm.at[pl.ds(row_base + (ch + 1) * CHUNK_ROWS, CHUNK_ROWS)],
                xbufs[1 - p], in_sems[1 - p])
        if out_copies[p] is not None:
            out_copies[p].wait()
        in_copies[p].wait()
        xv = xbufs[p]
        cntv = cbufs[p]

        @plsc.parallel_loop(0, CHUNK_ROWS, step=1, unroll=2)
        def _row(r, xv=xv, cntv=cntv):
            cntv[r, pl.ds(0, 16)] = zeros
            ridx = jnp.full((16,), r, dtype=jnp.int32)
            for k in range(FULL_VECS):
                vals = xv[r, pl.ds(k * 16, 16)]
                plsc.addupdate_scatter(cntv, [ridx, vals], ones)
            tail = xv[r, pl.ds(TAIL_OFF, 16)]
            plsc.addupdate_scatter(cntv, [ridx, tail], ones, mask=tail_mask)

        out_copies[p] = pltpu.async_copy(
            cntv, cnt_hbm.at[pl.ds(row_base + ch * CHUNK_ROWS, CHUNK_ROWS)],
            out_sems[p])

    for p in range(2):
        if out_copies[p] is not None:
            out_copies[p].wait()


def _sc_histogram(x):
    mesh = plsc.VectorSubcoreMesh(core_axis_name="c", subcore_axis_name="s")
    cp = pltpu.CompilerParams(use_tc_tiling_on_sc=True)
    if "needs_layout_passes" in pltpu.CompilerParams.__dataclass_fields__:
        cp = dataclasses.replace(cp, needs_layout_passes=False)
    f = pl.kernel(
        _sc_hist_kernel,
        out_type=jax.ShapeDtypeStruct((B, 16), jnp.float32),
        mesh=mesh,
        scratch_types=[
            pltpu.VMEM((CHUNK_ROWS, L), jnp.int32),
            pltpu.VMEM((CHUNK_ROWS, L), jnp.int32),
            pltpu.VMEM((CHUNK_ROWS, 16), jnp.float32),
            pltpu.VMEM((CHUNK_ROWS, 16), jnp.float32),
            pltpu.SemaphoreType.DMA,
            pltpu.SemaphoreType.DMA,
            pltpu.SemaphoreType.DMA,
            pltpu.SemaphoreType.DMA,
        ],
        compiler_params=cp,
    )
    return f(x)


MLP_BLK = 2048


def _mlp_kernel(cnt_ref, table_ref, w1_ref, b1_ref, w2_ref, b2_ref, out_ref):
    m = jnp.dot(table_ref[...], w1_ref[...].T,
                preferred_element_type=jnp.float32) * (1.0 / L)
    mp = jnp.concatenate([m, jnp.zeros((16 - VOCAB, H), jnp.float32)], axis=0)
    cnt = cnt_ref[...]
    h = jnp.maximum(
        jnp.dot(cnt, mp, preferred_element_type=jnp.float32) + b1_ref[0][None, :],
        0.0)
    out_ref[...] = (jnp.dot(h, w2_ref[...].T, preferred_element_type=jnp.float32)
                    + b2_ref[0][None, :])


def _tc_mlp(counts, table, W1, b1, W2, b2):
    return pl.pallas_call(
        _mlp_kernel,
        grid=(B // MLP_BLK,),
        in_specs=[
            pl.BlockSpec((MLP_BLK, 16), lambda i: (i, 0)),
            pl.BlockSpec((VOCAB, E), lambda i: (0, 0)),
            pl.BlockSpec((H, E), lambda i: (0, 0)),
            pl.BlockSpec((1, H), lambda i: (0, 0)),
            pl.BlockSpec((O, H), lambda i: (0, 0)),
            pl.BlockSpec((1, O), lambda i: (0, 0)),
        ],
        out_specs=pl.BlockSpec((MLP_BLK, O), lambda i: (i, 0)),
        out_shape=jax.ShapeDtypeStruct((B, O), jnp.float32),
        compiler_params=pltpu.CompilerParams(
            dimension_semantics=("arbitrary",),
        ),
    )(counts, table, W1, b1.reshape(1, H), W2, b2.reshape(1, O))


def kernel(x, table, W1, b1, W2, b2):
    counts = _sc_histogram(x.astype(jnp.int32))
    return _tc_mlp(counts, table, W1, b1, W2, b2)
